# Initial kernel scaffold; baseline (speedup 1.0000x reference)
import jax
import jax.numpy as jnp
from jax.experimental import pallas as pl


def kernel(inputs):
    raise NotImplementedError("write your pallas kernel here")


# speedup vs baseline: 85.4307x; 85.4307x over previous
import functools
import math

import jax
import jax.numpy as jnp
from jax.experimental import pallas as pl
from jax.experimental.pallas import tpu as pltpu

_ACT_SPARSITY = 0.65
_BLOCK_ROWS = 256


def _topk_mask_block(x_ref, o_ref, *, k):
    x = x_ref[...]
    bits = jax.lax.bitcast_convert_type(x, jnp.uint32)
    sign = (bits >> jnp.uint32(31)) == jnp.uint32(1)
    u = jnp.where(sign, ~bits, bits | jnp.uint32(0x80000000))

    def body(i, carry):
        del i
        thresh, bit = carry
        cand = thresh | bit
        cnt = jnp.sum((u >= cand).astype(jnp.int32), axis=1, keepdims=True)
        thresh = jnp.where(cnt >= k, cand, thresh)
        return thresh, bit >> jnp.uint32(1)

    rows = x.shape[0]
    thresh0 = jnp.zeros((rows, 1), jnp.uint32)
    bit0 = jnp.full((rows, 1), jnp.uint32(0x80000000))
    thresh, _ = jax.lax.fori_loop(0, 32, body, (thresh0, bit0))
    o_ref[...] = jnp.where(u >= thresh, x, 0.0)


def kernel(inputs):
    n, f = inputs.shape
    k = math.floor((1.0 - _ACT_SPARSITY) * f)
    grid = n // _BLOCK_ROWS
    return pl.pallas_call(
        functools.partial(_topk_mask_block, k=k),
        grid=(grid,),
        in_specs=[pl.BlockSpec((_BLOCK_ROWS, f), lambda i: (i, 0))],
        out_specs=pl.BlockSpec((_BLOCK_ROWS, f), lambda i: (i, 0)),
        out_shape=jax.ShapeDtypeStruct((n, f), inputs.dtype),
        compiler_params=pltpu.CompilerParams(
            dimension_semantics=("arbitrary",),
        ),
    )(inputs)
